# baseline (speedup 1.0000x reference)
import functools

import jax
import jax.numpy as jnp
from jax import lax
from jax.experimental import pallas as pl
from jax.experimental.pallas import tpu as pltpu
from jax.experimental.pallas import tpu_sc as plsc

B, T = 1024, 200
N = B * T
V1, V2, V3 = 100000, 100000, 1000000
D1, D2, D3 = 32, 32, 16
DO = D1 + D2 + D3 + 1
EPS = 1e-5

NC, NS = 2, 16
NW = NC * NS
ROWS_W = N // NW
SUB = 128
KSUB = 5
CH = SUB * KSUB
NCH = ROWS_W // CH


def _scaler_body(a_ref, o_ref):
    x = a_ref[...]
    mean = jnp.mean(x)
    cx = x - mean
    var = jnp.mean(cx * cx)
    y = cx * lax.rsqrt(var + EPS)
    o_ref[...] = jnp.log1p(jnp.abs(y)) * jnp.sign(y)


def _transpose_table(wt, v_block):
    d, v = wt.shape
    g = v_block * d // 128
    grid = (v + v_block - 1) // v_block

    def body(x_ref, o_ref):
        x = x_ref[...]
        x = x.reshape(d, g, 128 // d)
        o_ref[...] = x.transpose(1, 2, 0).reshape(g, 128)

    return pl.pallas_call(
        body,
        grid=(grid,),
        in_specs=[pl.BlockSpec((d, v_block), lambda i: (0, i))],
        out_specs=pl.BlockSpec((g, 128), lambda i: (i, 0)),
        out_shape=jax.ShapeDtypeStruct((v * d // 128, 128), jnp.float32),
    )(wt)


_mesh = plsc.VectorSubcoreMesh(core_axis_name="c", subcore_axis_name="s")


@functools.partial(
    pl.kernel,
    mesh=_mesh,
    compiler_params=pltpu.CompilerParams(use_tc_tiling_on_sc=False),
    out_type=jax.ShapeDtypeStruct((N * DO,), jnp.float32),
    scratch_types=[
        pltpu.VMEM((CH,), jnp.int32),
        pltpu.VMEM((CH,), jnp.int32),
        pltpu.VMEM((CH,), jnp.int32),
        pltpu.VMEM((KSUB, SUB, D1), jnp.float32),
        pltpu.VMEM((KSUB, SUB, D2), jnp.float32),
        pltpu.VMEM((KSUB, SUB, D3), jnp.float32),
        pltpu.VMEM((CH,), jnp.float32),
        pltpu.VMEM((CH * DO,), jnp.float32),
        pltpu.SemaphoreType.DMA,
    ],
)
def _sc_gather(mcc_hbm, tr_hbm, mer_hbm, num_hbm, wm_hbm, wt_hbm,
               we_hbm, out_hbm, idx1, idx2, idx3, r1, r2, r3, numv, comb, sem):
    wid = lax.axis_index("s") * NC + lax.axis_index("c")

    def body(c, carry):
        base = wid * ROWS_W + c * CH
        pltpu.sync_copy(mcc_hbm.at[pl.ds(base, CH)], idx1)
        pltpu.sync_copy(tr_hbm.at[pl.ds(base, CH)], idx2)
        pltpu.sync_copy(mer_hbm.at[pl.ds(base, CH)], idx3)
        pltpu.sync_copy(num_hbm.at[pl.ds(base, CH)], numv)
        cps = []
        for j in range(KSUB):
            sl = pl.ds(j * SUB, SUB)
            cps.append(pltpu.async_copy(
                wm_hbm.at[idx1.at[sl]], r1.at[j], sem))
            cps.append(pltpu.async_copy(
                wt_hbm.at[idx2.at[sl]], r2.at[j], sem))
            cps.append(pltpu.async_copy(
                we_hbm.at[idx3.at[sl]], r3.at[j], sem))
        for cp in cps:
            cp.wait()

        def interleave(k, carry2):
            numvec = numv[pl.ds(k * 16, 16)]
            j = k // (SUB // 16)
            kk = k % (SUB // 16)
            for t in range(16):
                i = kk * 16 + t
                o = (k * 16 + t) * DO
                comb[pl.ds(o, 16)] = r1[j, i, pl.ds(0, 16)]
                comb[pl.ds(o + 16, 16)] = r1[j, i, pl.ds(16, 16)]
                comb[pl.ds(o + 32, 16)] = r2[j, i, pl.ds(0, 16)]
                comb[pl.ds(o + 48, 16)] = r2[j, i, pl.ds(16, 16)]
                comb[pl.ds(o + 65, 16)] = jnp.broadcast_to(numvec[t], (16,))
                comb[pl.ds(o + 64, 16)] = r3[j, i, pl.ds(0, 16)]
            return carry2

        lax.fori_loop(0, CH // 16, interleave, 0)
        pltpu.sync_copy(comb, out_hbm.at[pl.ds(base * DO, CH * DO)])
        return carry

    lax.fori_loop(0, NCH, body, 0)


def kernel(mcc_code, tr_type, merchant_id, amount, seq_lens, W_mcc, W_tr, W_mer):
    del seq_lens
    num = pl.pallas_call(
        _scaler_body,
        out_shape=jax.ShapeDtypeStruct((B, T), jnp.float32),
    )(amount)
    wm = _transpose_table(W_mcc.T, 2048).reshape(V1, D1)
    wt = _transpose_table(W_tr.T, 2048).reshape(V2, D2)
    we = _transpose_table(W_mer.T, 4096).reshape(V3, D3)
    mcc1d = mcc_code.astype(jnp.int32).reshape(N)
    tr1d = tr_type.astype(jnp.int32).reshape(N)
    mer1d = merchant_id.astype(jnp.int32).reshape(N)
    out = _sc_gather(mcc1d, tr1d, mer1d, num.reshape(N), wm, wt, we)
    return out.reshape(B, T, DO)


# speedup vs baseline: 1.8028x; 1.8028x over previous
import functools

import jax
import jax.numpy as jnp
from jax import lax
from jax.experimental import pallas as pl
from jax.experimental.pallas import tpu as pltpu
from jax.experimental.pallas import tpu_sc as plsc

B, T = 1024, 200
N = B * T
V1, V2, V3 = 100000, 100000, 1000000
D1, D2, D3 = 32, 32, 16
DO = D1 + D2 + D3 + 1
EPS = 1e-5
SLOT = 96
TT = T // 8
RW = 8 * SLOT

NC, NS = 2, 16
NW = NC * NS
BPW = B // NW
SUBA, SUBB = 128, 72


def _scaler_body(a_ref, o_ref):
    x = a_ref[...]
    mean = jnp.mean(x)
    cx = x - mean
    var = jnp.mean(cx * cx)
    y = cx * lax.rsqrt(var + EPS)
    o_ref[...] = jnp.log1p(jnp.abs(y)) * jnp.sign(y)


def _repack_body(x_ref, o_ref):
    for s in range(8):
        o_ref[:, 0, 0, s, :] = x_ref[:, pl.ds(s * SLOT, DO)].T


_mesh = plsc.VectorSubcoreMesh(core_axis_name="c", subcore_axis_name="s")


@functools.partial(
    pl.kernel,
    mesh=_mesh,
    compiler_params=pltpu.CompilerParams(use_tc_tiling_on_sc=False),
    out_type=jax.ShapeDtypeStruct((TT * B, RW), jnp.float32),
    scratch_types=[
        pltpu.VMEM((208,), jnp.int32),
        pltpu.VMEM((208,), jnp.int32),
        pltpu.VMEM((208,), jnp.int32),
        pltpu.VMEM((SUBA, D1), jnp.float32),
        pltpu.VMEM((SUBB, D1), jnp.float32),
        pltpu.VMEM((SUBA, D2), jnp.float32),
        pltpu.VMEM((SUBB, D2), jnp.float32),
        pltpu.VMEM((SUBA, D3), jnp.float32),
        pltpu.VMEM((SUBB, D3), jnp.float32),
        pltpu.VMEM((208,), jnp.float32),
        pltpu.VMEM((TT, RW), jnp.float32),
        pltpu.SemaphoreType.DMA,
        pltpu.SemaphoreType.DMA,
    ],
)
def _sc_gather(mcc_hbm, tr_hbm, mer_hbm, num_hbm, wm_hbm, wt_hbm, we_hbm,
               out_hbm, idx1, idx2, idx3, r1a, r1b, r2a, r2b, r3a, r3b,
               numv, comb, sem, osem):
    wid = lax.axis_index("s") * NC + lax.axis_index("c")

    def body(bi, carry):
        b = wid * BPW + bi
        base = b * T
        pltpu.sync_copy(mcc_hbm.at[pl.ds(base, T)], idx1.at[pl.ds(0, T)])
        pltpu.sync_copy(tr_hbm.at[pl.ds(base, T)], idx2.at[pl.ds(0, T)])
        pltpu.sync_copy(mer_hbm.at[pl.ds(base, T)], idx3.at[pl.ds(0, T)])
        pltpu.sync_copy(num_hbm.at[pl.ds(base, T)], numv.at[pl.ds(0, T)])
        cps = [
            pltpu.async_copy(wm_hbm.at[idx1.at[pl.ds(0, SUBA)]], r1a, sem),
            pltpu.async_copy(wt_hbm.at[idx2.at[pl.ds(0, SUBA)]], r2a, sem),
            pltpu.async_copy(we_hbm.at[idx3.at[pl.ds(0, SUBA)]], r3a, sem),
            pltpu.async_copy(wm_hbm.at[idx1.at[pl.ds(SUBA, SUBB)]], r1b, sem),
            pltpu.async_copy(wt_hbm.at[idx2.at[pl.ds(SUBA, SUBB)]], r2b, sem),
            pltpu.async_copy(we_hbm.at[idx3.at[pl.ds(SUBA, SUBB)]], r3b, sem),
        ]
        for cp in cps:
            cp.wait()

        def emit(g, row0, x1, x2, x3):
            numvec = numv[pl.ds(g * 8, 16)]
            for j in range(8):
                i = row0 + j
                o = j * SLOT
                comb[g, pl.ds(o, 16)] = x1[i, pl.ds(0, 16)]
                comb[g, pl.ds(o + 16, 16)] = x1[i, pl.ds(16, 16)]
                comb[g, pl.ds(o + 32, 16)] = x2[i, pl.ds(0, 16)]
                comb[g, pl.ds(o + 48, 16)] = x2[i, pl.ds(16, 16)]
                comb[g, pl.ds(o + 65, 16)] = jnp.broadcast_to(numvec[j], (16,))
                comb[g, pl.ds(o + 64, 16)] = x3[i, pl.ds(0, 16)]

        def grp_a(g, carry2):
            emit(g, g * 8, r1a, r2a, r3a)
            return carry2

        def grp_b(g, carry2):
            emit(g, g * 8 - SUBA, r1b, r2b, r3b)
            return carry2

        lax.fori_loop(0, SUBA // 8, grp_a, 0)
        lax.fori_loop(SUBA // 8, TT, grp_b, 0)

        ocps = []
        for ti in range(TT):
            ocps.append(pltpu.async_copy(
                comb.at[pl.ds(ti, 1)],
                out_hbm.at[pl.ds(ti * B + b, 1)], osem))
        for cp in ocps:
            cp.wait()
        return carry

    lax.fori_loop(0, BPW, body, 0)


def kernel(mcc_code, tr_type, merchant_id, amount, seq_lens, W_mcc, W_tr, W_mer):
    del seq_lens
    num = pl.pallas_call(
        _scaler_body,
        out_shape=jax.ShapeDtypeStruct((B, T), jnp.float32),
    )(amount)
    mcc1d = mcc_code.astype(jnp.int32).reshape(N)
    tr1d = tr_type.astype(jnp.int32).reshape(N)
    mer1d = merchant_id.astype(jnp.int32).reshape(N)
    stg = _sc_gather(mcc1d, tr1d, mer1d, num.reshape(N), W_mcc, W_tr, W_mer)
    planes = pl.pallas_call(
        _repack_body,
        grid=(TT, B // 128),
        in_specs=[pl.BlockSpec((128, RW), lambda ti, bj: (ti * (B // 128) + bj, 0))],
        out_specs=pl.BlockSpec((DO, 1, 1, 8, 128), lambda ti, bj: (0, ti, bj, 0, 0)),
        out_shape=jax.ShapeDtypeStruct((DO, TT, B // 128, 8, 128), jnp.float32),
    )(stg)
    return planes.transpose(2, 4, 1, 3, 0).reshape(B, T, DO)


# speedup vs baseline: 1.9065x; 1.0575x over previous
import functools

import jax
import jax.numpy as jnp
from jax import lax
from jax.experimental import pallas as pl
from jax.experimental.pallas import tpu as pltpu
from jax.experimental.pallas import tpu_sc as plsc

B, T = 1024, 200
N = B * T
V1, V2, V3 = 100000, 100000, 1000000
D1, D2, D3 = 32, 32, 16
DO = D1 + D2 + D3 + 1
EPS = 1e-5
SLOT = 96
TT = T // 8
RW = 8 * SLOT

NC, NS = 2, 16
NW = NC * NS
BPW = B // NW
SUBA, SUBB = 128, 72


def _scaler_body(a_ref, o_ref):
    x = a_ref[...]
    mean = jnp.mean(x)
    cx = x - mean
    var = jnp.mean(cx * cx)
    y = cx * lax.rsqrt(var + EPS)
    o_ref[...] = jnp.log1p(jnp.abs(y)) * jnp.sign(y)


def _repack_body(x_ref, o_ref):
    for s in range(8):
        o_ref[:, 0, 0, s, :] = x_ref[:, pl.ds(s * SLOT, DO)].T


_mesh = plsc.VectorSubcoreMesh(core_axis_name="c", subcore_axis_name="s")


@functools.partial(
    pl.kernel,
    mesh=_mesh,
    compiler_params=pltpu.CompilerParams(use_tc_tiling_on_sc=False),
    out_type=jax.ShapeDtypeStruct((TT * B, RW), jnp.float32),
    scratch_types=[
        pltpu.VMEM((2, 208), jnp.int32),
        pltpu.VMEM((2, 208), jnp.int32),
        pltpu.VMEM((2, 208), jnp.int32),
        pltpu.VMEM((2, SUBA, D1), jnp.float32),
        pltpu.VMEM((2, SUBB, D1), jnp.float32),
        pltpu.VMEM((2, SUBA, D2), jnp.float32),
        pltpu.VMEM((2, SUBB, D2), jnp.float32),
        pltpu.VMEM((2, SUBA, D3), jnp.float32),
        pltpu.VMEM((2, SUBB, D3), jnp.float32),
        pltpu.VMEM((2, 208), jnp.float32),
        pltpu.VMEM((2, TT, RW), jnp.float32),
        pltpu.SemaphoreType.DMA,
        pltpu.SemaphoreType.DMA,
        pltpu.SemaphoreType.DMA,
        pltpu.SemaphoreType.DMA,
    ],
)
def _sc_gather(mcc_hbm, tr_hbm, mer_hbm, num_hbm, wm_hbm, wt_hbm, we_hbm,
               out_hbm, idx1, idx2, idx3, r1a, r1b, r2a, r2b, r3a, r3b,
               numv, comb, sem0, sem1, osem0, osem1):
    wid = lax.axis_index("s") * NC + lax.axis_index("c")
    sems = (sem0, sem1)
    osems = (osem0, osem1)

    def gather_cps(bi, u):
        del bi
        sem = sems[u]
        return [
            pltpu.make_async_copy(
                wm_hbm.at[idx1.at[u, pl.ds(0, SUBA)]], r1a.at[u], sem),
            pltpu.make_async_copy(
                wt_hbm.at[idx2.at[u, pl.ds(0, SUBA)]], r2a.at[u], sem),
            pltpu.make_async_copy(
                we_hbm.at[idx3.at[u, pl.ds(0, SUBA)]], r3a.at[u], sem),
            pltpu.make_async_copy(
                wm_hbm.at[idx1.at[u, pl.ds(SUBA, SUBB)]], r1b.at[u], sem),
            pltpu.make_async_copy(
                wt_hbm.at[idx2.at[u, pl.ds(SUBA, SUBB)]], r2b.at[u], sem),
            pltpu.make_async_copy(
                we_hbm.at[idx3.at[u, pl.ds(SUBA, SUBB)]], r3b.at[u], sem),
        ]

    def stage_and_fire(bi, u):
        base = (wid * BPW + bi) * T
        pltpu.sync_copy(mcc_hbm.at[pl.ds(base, T)], idx1.at[u, pl.ds(0, T)])
        pltpu.sync_copy(tr_hbm.at[pl.ds(base, T)], idx2.at[u, pl.ds(0, T)])
        pltpu.sync_copy(mer_hbm.at[pl.ds(base, T)], idx3.at[u, pl.ds(0, T)])
        pltpu.sync_copy(num_hbm.at[pl.ds(base, T)], numv.at[u, pl.ds(0, T)])
        for cp in gather_cps(bi, u):
            cp.start()

    def out_cps(bi, u):
        b = wid * BPW + bi
        return [
            pltpu.make_async_copy(
                comb.at[u, pl.ds(ti, 1)],
                out_hbm.at[pl.ds(ti * B + b, 1)], osems[u])
            for ti in range(TT)
        ]

    def process(bi, u):
        for cp in gather_cps(bi, u):
            cp.wait()

        def emit(g, row0, x1, x2, x3):
            numvec = numv[u, pl.ds(g * 8, 16)]
            for j in range(8):
                i = row0 + j
                o = j * SLOT
                comb[u, g, pl.ds(o, 16)] = x1[u, i, pl.ds(0, 16)]
                comb[u, g, pl.ds(o + 16, 16)] = x1[u, i, pl.ds(16, 16)]
                comb[u, g, pl.ds(o + 32, 16)] = x2[u, i, pl.ds(0, 16)]
                comb[u, g, pl.ds(o + 48, 16)] = x2[u, i, pl.ds(16, 16)]
                comb[u, g, pl.ds(o + 65, 16)] = jnp.broadcast_to(
                    numvec[j], (16,))
                comb[u, g, pl.ds(o + 64, 16)] = x3[u, i, pl.ds(0, 16)]

        def grp_a(g, carry2):
            emit(g, g * 8, r1a, r2a, r3a)
            return carry2

        def grp_b(g, carry2):
            emit(g, g * 8 - SUBA, r1b, r2b, r3b)
            return carry2

        lax.fori_loop(0, SUBA // 8, grp_a, 0)
        lax.fori_loop(SUBA // 8, TT, grp_b, 0)
        for cp in out_cps(bi, u):
            cp.start()

    stage_and_fire(0, 0)

    def body(bi, carry):
        even = lax.rem(bi, 2) == 0
        more = bi + 1 < BPW

        @pl.when(jnp.logical_and(more, even))
        def _():
            stage_and_fire(bi + 1, 1)

        @pl.when(jnp.logical_and(more, jnp.logical_not(even)))
        def _():
            stage_and_fire(bi + 1, 0)

        @pl.when(even)
        def _():
            @pl.when(bi >= 2)
            def _():
                for cp in out_cps(bi - 2, 0):
                    cp.wait()
            process(bi, 0)

        @pl.when(jnp.logical_not(even))
        def _():
            @pl.when(bi >= 2)
            def _():
                for cp in out_cps(bi - 2, 1):
                    cp.wait()
            process(bi, 1)

        return carry

    lax.fori_loop(0, BPW, body, 0)
    for cp in out_cps(BPW - 2, 0):
        cp.wait()
    for cp in out_cps(BPW - 1, 1):
        cp.wait()


def kernel(mcc_code, tr_type, merchant_id, amount, seq_lens, W_mcc, W_tr, W_mer):
    del seq_lens
    num = pl.pallas_call(
        _scaler_body,
        out_shape=jax.ShapeDtypeStruct((B, T), jnp.float32),
    )(amount)
    mcc1d = mcc_code.astype(jnp.int32).reshape(N)
    tr1d = tr_type.astype(jnp.int32).reshape(N)
    mer1d = merchant_id.astype(jnp.int32).reshape(N)
    stg = _sc_gather(mcc1d, tr1d, mer1d, num.reshape(N), W_mcc, W_tr, W_mer)
    planes = pl.pallas_call(
        _repack_body,
        grid=(TT, B // 128),
        in_specs=[pl.BlockSpec((128, RW), lambda ti, bj: (ti * (B // 128) + bj, 0))],
        out_specs=pl.BlockSpec((DO, 1, 1, 8, 128), lambda ti, bj: (0, ti, bj, 0, 0)),
        out_shape=jax.ShapeDtypeStruct((DO, TT, B // 128, 8, 128), jnp.float32),
    )(stg)
    return planes.transpose(2, 4, 1, 3, 0).reshape(B, T, DO)


# speedup vs baseline: 1.9930x; 1.0454x over previous
import functools

import jax
import jax.numpy as jnp
from jax import lax
from jax.experimental import pallas as pl
from jax.experimental.pallas import tpu as pltpu
from jax.experimental.pallas import tpu_sc as plsc

B, T = 1024, 200
N = B * T
V1, V2, V3 = 100000, 100000, 1000000
D1, D2, D3 = 32, 32, 16
DO = D1 + D2 + D3 + 1
EPS = 1e-5
SLOT = 96
TT = T // 8
RW = 8 * SLOT

NC, NS = 2, 16
NW = NC * NS
BPW = B // NW
SUBA, SUBB = 128, 72


def _scaler_body(a_ref, o_ref):
    x = a_ref[...]
    mean = jnp.mean(x)
    cx = x - mean
    var = jnp.mean(cx * cx)
    y = cx * lax.rsqrt(var + EPS)
    o_ref[...] = jnp.log1p(jnp.abs(y)) * jnp.sign(y)


def _repack_body(x_ref, o_ref):
    for s in range(8):
        o_ref[:, 0, 0, s, :] = x_ref[:, pl.ds(s * SLOT, DO)].T


_mesh = plsc.VectorSubcoreMesh(core_axis_name="c", subcore_axis_name="s")


@functools.partial(
    pl.kernel,
    mesh=_mesh,
    compiler_params=pltpu.CompilerParams(use_tc_tiling_on_sc=False),
    out_type=jax.ShapeDtypeStruct((TT * B, RW), jnp.float32),
    scratch_types=[
        pltpu.VMEM((2, 208), jnp.int32),
        pltpu.VMEM((2, 208), jnp.int32),
        pltpu.VMEM((2, 208), jnp.int32),
        pltpu.VMEM((2, SUBA, D1), jnp.float32),
        pltpu.VMEM((2, SUBB, D1), jnp.float32),
        pltpu.VMEM((2, SUBA, D2), jnp.float32),
        pltpu.VMEM((2, SUBB, D2), jnp.float32),
        pltpu.VMEM((2, SUBA, D3), jnp.float32),
        pltpu.VMEM((2, SUBB, D3), jnp.float32),
        pltpu.VMEM((2, 208), jnp.float32),
        pltpu.VMEM((2, TT, RW), jnp.float32),
        pltpu.SemaphoreType.DMA,
        pltpu.SemaphoreType.DMA,
        pltpu.SemaphoreType.DMA,
        pltpu.SemaphoreType.DMA,
    ],
)
def _sc_gather(mcc_hbm, tr_hbm, mer_hbm, num_hbm, wm_hbm, wt_hbm, we_hbm,
               out_hbm, idx1, idx2, idx3, r1a, r1b, r2a, r2b, r3a, r3b,
               numv, comb, sem0, sem1, osem0, osem1):
    wid = lax.axis_index("s") * NC + lax.axis_index("c")
    sems = (sem0, sem1)
    osems = (osem0, osem1)

    def gather_cps(bi, u):
        del bi
        sem = sems[u]
        return [
            pltpu.make_async_copy(
                wm_hbm.at[idx1.at[u, pl.ds(0, SUBA)]], r1a.at[u], sem),
            pltpu.make_async_copy(
                wt_hbm.at[idx2.at[u, pl.ds(0, SUBA)]], r2a.at[u], sem),
            pltpu.make_async_copy(
                we_hbm.at[idx3.at[u, pl.ds(0, SUBA)]], r3a.at[u], sem),
            pltpu.make_async_copy(
                wm_hbm.at[idx1.at[u, pl.ds(SUBA, SUBB)]], r1b.at[u], sem),
            pltpu.make_async_copy(
                wt_hbm.at[idx2.at[u, pl.ds(SUBA, SUBB)]], r2b.at[u], sem),
            pltpu.make_async_copy(
                we_hbm.at[idx3.at[u, pl.ds(SUBA, SUBB)]], r3b.at[u], sem),
        ]

    def stage_and_fire(bi, u):
        base = (wid * BPW + bi) * T
        stcps = [
            pltpu.make_async_copy(
                mcc_hbm.at[pl.ds(base, T)], idx1.at[u, pl.ds(0, T)], sems[u]),
            pltpu.make_async_copy(
                tr_hbm.at[pl.ds(base, T)], idx2.at[u, pl.ds(0, T)], sems[u]),
            pltpu.make_async_copy(
                mer_hbm.at[pl.ds(base, T)], idx3.at[u, pl.ds(0, T)], sems[u]),
            pltpu.make_async_copy(
                num_hbm.at[pl.ds(base, T)], numv.at[u, pl.ds(0, T)], sems[u]),
        ]
        for cp in stcps:
            cp.start()
        for cp in stcps:
            cp.wait()
        for cp in gather_cps(bi, u):
            cp.start()

    def out_cps(bi, u):
        b = wid * BPW + bi
        return [
            pltpu.make_async_copy(
                comb.at[u, pl.ds(ti, 1)],
                out_hbm.at[pl.ds(ti * B + b, 1)], osems[u])
            for ti in range(TT)
        ]

    def process(bi, u):
        for cp in gather_cps(bi, u):
            cp.wait()

        def emit(g, row0, x1, x2, x3):
            numvec = numv[u, pl.ds(g * 8, 16)]
            for j in range(8):
                i = row0 + j
                o = j * SLOT
                comb[u, g, pl.ds(o, 16)] = x1[u, i, pl.ds(0, 16)]
                comb[u, g, pl.ds(o + 16, 16)] = x1[u, i, pl.ds(16, 16)]
                comb[u, g, pl.ds(o + 32, 16)] = x2[u, i, pl.ds(0, 16)]
                comb[u, g, pl.ds(o + 48, 16)] = x2[u, i, pl.ds(16, 16)]
                comb[u, g, pl.ds(o + 65, 16)] = jnp.broadcast_to(
                    numvec[j], (16,))
                comb[u, g, pl.ds(o + 64, 16)] = x3[u, i, pl.ds(0, 16)]

        def grp_a(g, carry2):
            emit(g, g * 8, r1a, r2a, r3a)
            return carry2

        def grp_b(g, carry2):
            emit(g, g * 8 - SUBA, r1b, r2b, r3b)
            return carry2

        lax.fori_loop(0, SUBA // 8, grp_a, 0)
        lax.fori_loop(SUBA // 8, TT, grp_b, 0)
        for cp in out_cps(bi, u):
            cp.start()

    stage_and_fire(0, 0)

    def body(bi, carry):
        even = lax.rem(bi, 2) == 0
        more = bi + 1 < BPW

        @pl.when(jnp.logical_and(more, even))
        def _():
            stage_and_fire(bi + 1, 1)

        @pl.when(jnp.logical_and(more, jnp.logical_not(even)))
        def _():
            stage_and_fire(bi + 1, 0)

        @pl.when(even)
        def _():
            @pl.when(bi >= 2)
            def _():
                for cp in out_cps(bi - 2, 0):
                    cp.wait()
            process(bi, 0)

        @pl.when(jnp.logical_not(even))
        def _():
            @pl.when(bi >= 2)
            def _():
                for cp in out_cps(bi - 2, 1):
                    cp.wait()
            process(bi, 1)

        return carry

    lax.fori_loop(0, BPW, body, 0)
    for cp in out_cps(BPW - 2, 0):
        cp.wait()
    for cp in out_cps(BPW - 1, 1):
        cp.wait()


def kernel(mcc_code, tr_type, merchant_id, amount, seq_lens, W_mcc, W_tr, W_mer):
    del seq_lens
    num = pl.pallas_call(
        _scaler_body,
        out_shape=jax.ShapeDtypeStruct((B, T), jnp.float32),
    )(amount)
    mcc1d = mcc_code.astype(jnp.int32).reshape(N)
    tr1d = tr_type.astype(jnp.int32).reshape(N)
    mer1d = merchant_id.astype(jnp.int32).reshape(N)
    stg = _sc_gather(mcc1d, tr1d, mer1d, num.reshape(N), W_mcc, W_tr, W_mer)
    planes = pl.pallas_call(
        _repack_body,
        grid=(TT, B // 128),
        in_specs=[pl.BlockSpec((128, RW), lambda ti, bj: (ti * (B // 128) + bj, 0))],
        out_specs=pl.BlockSpec((DO, 1, 1, 8, 128), lambda ti, bj: (0, ti, bj, 0, 0)),
        out_shape=jax.ShapeDtypeStruct((DO, TT, B // 128, 8, 128), jnp.float32),
    )(stg)
    return planes.transpose(2, 4, 1, 3, 0).reshape(B, T, DO)


# speedup vs baseline: 2.1090x; 1.0582x over previous
import functools

import jax
import jax.numpy as jnp
from jax import lax
from jax.experimental import pallas as pl
from jax.experimental.pallas import tpu as pltpu
from jax.experimental.pallas import tpu_sc as plsc

B, T = 1024, 200
N = B * T
V1, V2, V3 = 100000, 100000, 1000000
D1, D2, D3 = 32, 32, 16
DO = D1 + D2 + D3 + 1
EPS = 1e-5
SLOT = 96
TT = T // 8
RW = 8 * SLOT

NC, NS = 2, 16
NW = NC * NS
BPW = B // NW
SUBA, SUBB = 128, 72


def _scaler_body(a_ref, o_ref):
    x = a_ref[...]
    mean = jnp.mean(x)
    cx = x - mean
    var = jnp.mean(cx * cx)
    y = cx * lax.rsqrt(var + EPS)
    o_ref[...] = jnp.log1p(jnp.abs(y)) * jnp.sign(y)


def _repack_body(x_ref, o_ref):
    for h in range(2):
        for s in range(8):
            o_ref[:, 0, h, s, :] = (
                x_ref[pl.ds(h * 128, 128), pl.ds(s * SLOT, DO)].T)


_mesh = plsc.VectorSubcoreMesh(core_axis_name="c", subcore_axis_name="s")


@functools.partial(
    pl.kernel,
    mesh=_mesh,
    compiler_params=pltpu.CompilerParams(use_tc_tiling_on_sc=False),
    out_type=jax.ShapeDtypeStruct((TT * B, RW), jnp.float32),
    scratch_types=[
        pltpu.VMEM((2, 208), jnp.int32),
        pltpu.VMEM((2, 208), jnp.int32),
        pltpu.VMEM((2, 208), jnp.int32),
        pltpu.VMEM((2, SUBA, D1), jnp.float32),
        pltpu.VMEM((2, SUBB, D1), jnp.float32),
        pltpu.VMEM((2, SUBA, D2), jnp.float32),
        pltpu.VMEM((2, SUBB, D2), jnp.float32),
        pltpu.VMEM((2, SUBA, D3), jnp.float32),
        pltpu.VMEM((2, SUBB, D3), jnp.float32),
        pltpu.VMEM((2, 208), jnp.float32),
        pltpu.VMEM((2, TT, RW), jnp.float32),
        pltpu.SemaphoreType.DMA,
        pltpu.SemaphoreType.DMA,
        pltpu.SemaphoreType.DMA,
        pltpu.SemaphoreType.DMA,
    ],
)
def _sc_gather(mcc_hbm, tr_hbm, mer_hbm, num_hbm, wm_hbm, wt_hbm, we_hbm,
               out_hbm, idx1, idx2, idx3, r1a, r1b, r2a, r2b, r3a, r3b,
               numv, comb, sem0, sem1, osem0, osem1):
    wid = lax.axis_index("s") * NC + lax.axis_index("c")
    sems = (sem0, sem1)
    osems = (osem0, osem1)

    def gather_cps(bi, u):
        del bi
        sem = sems[u]
        return [
            pltpu.make_async_copy(
                wm_hbm.at[idx1.at[u, pl.ds(0, SUBA)]], r1a.at[u], sem),
            pltpu.make_async_copy(
                wt_hbm.at[idx2.at[u, pl.ds(0, SUBA)]], r2a.at[u], sem),
            pltpu.make_async_copy(
                we_hbm.at[idx3.at[u, pl.ds(0, SUBA)]], r3a.at[u], sem),
            pltpu.make_async_copy(
                wm_hbm.at[idx1.at[u, pl.ds(SUBA, SUBB)]], r1b.at[u], sem),
            pltpu.make_async_copy(
                wt_hbm.at[idx2.at[u, pl.ds(SUBA, SUBB)]], r2b.at[u], sem),
            pltpu.make_async_copy(
                we_hbm.at[idx3.at[u, pl.ds(SUBA, SUBB)]], r3b.at[u], sem),
        ]

    def stage_and_fire(bi, u):
        base = (wid * BPW + bi) * T
        stcps = [
            pltpu.make_async_copy(
                mcc_hbm.at[pl.ds(base, T)], idx1.at[u, pl.ds(0, T)], sems[u]),
            pltpu.make_async_copy(
                tr_hbm.at[pl.ds(base, T)], idx2.at[u, pl.ds(0, T)], sems[u]),
            pltpu.make_async_copy(
                mer_hbm.at[pl.ds(base, T)], idx3.at[u, pl.ds(0, T)], sems[u]),
            pltpu.make_async_copy(
                num_hbm.at[pl.ds(base, T)], numv.at[u, pl.ds(0, T)], sems[u]),
        ]
        for cp in stcps:
            cp.start()
        for cp in stcps:
            cp.wait()
        for cp in gather_cps(bi, u):
            cp.start()

    def out_cps(bi, u):
        b = wid * BPW + bi
        return [
            pltpu.make_async_copy(
                comb.at[u, pl.ds(ti, 1)],
                out_hbm.at[pl.ds(ti * B + b, 1)], osems[u])
            for ti in range(TT)
        ]

    def process(bi, u):
        for cp in gather_cps(bi, u):
            cp.wait()

        def emit(g, row0, x1, x2, x3):
            numvec = numv[u, pl.ds(g * 8, 16)]
            for j in range(8):
                i = row0 + j
                o = j * SLOT
                comb[u, g, pl.ds(o, 16)] = x1[u, i, pl.ds(0, 16)]
                comb[u, g, pl.ds(o + 16, 16)] = x1[u, i, pl.ds(16, 16)]
                comb[u, g, pl.ds(o + 32, 16)] = x2[u, i, pl.ds(0, 16)]
                comb[u, g, pl.ds(o + 48, 16)] = x2[u, i, pl.ds(16, 16)]
                comb[u, g, pl.ds(o + 65, 16)] = jnp.broadcast_to(
                    numvec[j], (16,))
                comb[u, g, pl.ds(o + 64, 16)] = x3[u, i, pl.ds(0, 16)]

        def grp_a(g, carry2):
            emit(g, g * 8, r1a, r2a, r3a)
            return carry2

        def grp_b(g, carry2):
            emit(g, g * 8 - SUBA, r1b, r2b, r3b)
            return carry2

        lax.fori_loop(0, SUBA // 8, grp_a, 0)
        lax.fori_loop(SUBA // 8, TT, grp_b, 0)
        for cp in out_cps(bi, u):
            cp.start()

    stage_and_fire(0, 0)

    def body(bi, carry):
        even = lax.rem(bi, 2) == 0
        more = bi + 1 < BPW

        @pl.when(jnp.logical_and(more, even))
        def _():
            stage_and_fire(bi + 1, 1)

        @pl.when(jnp.logical_and(more, jnp.logical_not(even)))
        def _():
            stage_and_fire(bi + 1, 0)

        @pl.when(even)
        def _():
            @pl.when(bi >= 2)
            def _():
                for cp in out_cps(bi - 2, 0):
                    cp.wait()
            process(bi, 0)

        @pl.when(jnp.logical_not(even))
        def _():
            @pl.when(bi >= 2)
            def _():
                for cp in out_cps(bi - 2, 1):
                    cp.wait()
            process(bi, 1)

        return carry

    lax.fori_loop(0, BPW, body, 0)
    for cp in out_cps(BPW - 2, 0):
        cp.wait()
    for cp in out_cps(BPW - 1, 1):
        cp.wait()


def kernel(mcc_code, tr_type, merchant_id, amount, seq_lens, W_mcc, W_tr, W_mer):
    del seq_lens
    num = pl.pallas_call(
        _scaler_body,
        out_shape=jax.ShapeDtypeStruct((B, T), jnp.float32),
    )(amount)
    mcc1d = mcc_code.astype(jnp.int32).reshape(N)
    tr1d = tr_type.astype(jnp.int32).reshape(N)
    mer1d = merchant_id.astype(jnp.int32).reshape(N)
    stg = _sc_gather(mcc1d, tr1d, mer1d, num.reshape(N), W_mcc, W_tr, W_mer)
    planes = pl.pallas_call(
        _repack_body,
        grid=(TT, B // 256),
        in_specs=[pl.BlockSpec((256, RW), lambda ti, bj: (ti * (B // 256) + bj, 0))],
        out_specs=pl.BlockSpec((DO, 1, 2, 8, 128), lambda ti, bj: (0, ti, bj, 0, 0)),
        out_shape=jax.ShapeDtypeStruct((DO, TT, B // 128, 8, 128), jnp.float32),
    )(stg)
    return planes.transpose(2, 4, 1, 3, 0).reshape(B, T, DO)


# speedup vs baseline: 2.1526x; 1.0207x over previous
import functools

import jax
import jax.numpy as jnp
from jax import lax
from jax.experimental import pallas as pl
from jax.experimental.pallas import tpu as pltpu
from jax.experimental.pallas import tpu_sc as plsc

B, T = 1024, 200
N = B * T
V1, V2, V3 = 100000, 100000, 1000000
D1, D2, D3 = 32, 32, 16
DO = D1 + D2 + D3 + 1
EPS = 1e-5
SLOT = 96
TT = T // 8
RW = 8 * SLOT

NC, NS = 2, 16
NW = NC * NS
BPW = B // NW
SUBA, SUBB = 128, 72


def _scaler_body(a_ref, o_ref):
    x = a_ref[...]
    mean = jnp.mean(x)
    cx = x - mean
    var = jnp.mean(cx * cx)
    y = cx * lax.rsqrt(var + EPS)
    o_ref[...] = jnp.log1p(jnp.abs(y)) * jnp.sign(y)


def _repack_body(x_ref, o_ref):
    for h in range(4):
        for s in range(8):
            o_ref[:, 0, h, s, :] = (
                x_ref[pl.ds(h * 128, 128), pl.ds(s * SLOT, DO)].T)


_mesh = plsc.VectorSubcoreMesh(core_axis_name="c", subcore_axis_name="s")


@functools.partial(
    pl.kernel,
    mesh=_mesh,
    compiler_params=pltpu.CompilerParams(use_tc_tiling_on_sc=False),
    out_type=jax.ShapeDtypeStruct((TT * B, RW), jnp.float32),
    scratch_types=[
        pltpu.VMEM((2, 208), jnp.int32),
        pltpu.VMEM((2, 208), jnp.int32),
        pltpu.VMEM((2, 208), jnp.int32),
        pltpu.VMEM((2, SUBA, D1), jnp.float32),
        pltpu.VMEM((2, SUBB, D1), jnp.float32),
        pltpu.VMEM((2, SUBA, D2), jnp.float32),
        pltpu.VMEM((2, SUBB, D2), jnp.float32),
        pltpu.VMEM((2, SUBA, D3), jnp.float32),
        pltpu.VMEM((2, SUBB, D3), jnp.float32),
        pltpu.VMEM((2, 208), jnp.float32),
        pltpu.VMEM((2, TT, RW), jnp.float32),
        pltpu.SemaphoreType.DMA,
        pltpu.SemaphoreType.DMA,
        pltpu.SemaphoreType.DMA,
        pltpu.SemaphoreType.DMA,
    ],
)
def _sc_gather(mcc_hbm, tr_hbm, mer_hbm, num_hbm, wm_hbm, wt_hbm, we_hbm,
               out_hbm, idx1, idx2, idx3, r1a, r1b, r2a, r2b, r3a, r3b,
               numv, comb, sem0, sem1, osem0, osem1):
    wid = lax.axis_index("s") * NC + lax.axis_index("c")
    sems = (sem0, sem1)
    osems = (osem0, osem1)

    def gather_cps(bi, u):
        del bi
        sem = sems[u]
        return [
            pltpu.make_async_copy(
                wm_hbm.at[idx1.at[u, pl.ds(0, SUBA)]], r1a.at[u], sem),
            pltpu.make_async_copy(
                wt_hbm.at[idx2.at[u, pl.ds(0, SUBA)]], r2a.at[u], sem),
            pltpu.make_async_copy(
                we_hbm.at[idx3.at[u, pl.ds(0, SUBA)]], r3a.at[u], sem),
            pltpu.make_async_copy(
                wm_hbm.at[idx1.at[u, pl.ds(SUBA, SUBB)]], r1b.at[u], sem),
            pltpu.make_async_copy(
                wt_hbm.at[idx2.at[u, pl.ds(SUBA, SUBB)]], r2b.at[u], sem),
            pltpu.make_async_copy(
                we_hbm.at[idx3.at[u, pl.ds(SUBA, SUBB)]], r3b.at[u], sem),
        ]

    def stage_and_fire(bi, u):
        base = (wid * BPW + bi) * T
        stcps = [
            pltpu.make_async_copy(
                mcc_hbm.at[pl.ds(base, T)], idx1.at[u, pl.ds(0, T)], sems[u]),
            pltpu.make_async_copy(
                tr_hbm.at[pl.ds(base, T)], idx2.at[u, pl.ds(0, T)], sems[u]),
            pltpu.make_async_copy(
                mer_hbm.at[pl.ds(base, T)], idx3.at[u, pl.ds(0, T)], sems[u]),
            pltpu.make_async_copy(
                num_hbm.at[pl.ds(base, T)], numv.at[u, pl.ds(0, T)], sems[u]),
        ]
        for cp in stcps:
            cp.start()
        for cp in stcps:
            cp.wait()
        for cp in gather_cps(bi, u):
            cp.start()

    def out_cps(bi, u):
        b = wid * BPW + bi
        return [
            pltpu.make_async_copy(
                comb.at[u, pl.ds(ti, 1)],
                out_hbm.at[pl.ds(ti * B + b, 1)], osems[u])
            for ti in range(TT)
        ]

    def process(bi, u):
        for cp in gather_cps(bi, u):
            cp.wait()

        def emit(g, row0, x1, x2, x3):
            numvec = numv[u, pl.ds(g * 8, 16)]
            for j in range(8):
                i = row0 + j
                o = j * SLOT
                comb[u, g, pl.ds(o, 16)] = x1[u, i, pl.ds(0, 16)]
                comb[u, g, pl.ds(o + 16, 16)] = x1[u, i, pl.ds(16, 16)]
                comb[u, g, pl.ds(o + 32, 16)] = x2[u, i, pl.ds(0, 16)]
                comb[u, g, pl.ds(o + 48, 16)] = x2[u, i, pl.ds(16, 16)]
                comb[u, g, pl.ds(o + 65, 16)] = jnp.broadcast_to(
                    numvec[j], (16,))
                comb[u, g, pl.ds(o + 64, 16)] = x3[u, i, pl.ds(0, 16)]

        def grp_a(g, carry2):
            emit(g, g * 8, r1a, r2a, r3a)
            return carry2

        def grp_b(g, carry2):
            emit(g, g * 8 - SUBA, r1b, r2b, r3b)
            return carry2

        lax.fori_loop(0, SUBA // 8, grp_a, 0)
        lax.fori_loop(SUBA // 8, TT, grp_b, 0)
        for cp in out_cps(bi, u):
            cp.start()

    stage_and_fire(0, 0)

    def body(bi, carry):
        even = lax.rem(bi, 2) == 0
        more = bi + 1 < BPW

        @pl.when(jnp.logical_and(more, even))
        def _():
            stage_and_fire(bi + 1, 1)

        @pl.when(jnp.logical_and(more, jnp.logical_not(even)))
        def _():
            stage_and_fire(bi + 1, 0)

        @pl.when(even)
        def _():
            @pl.when(bi >= 2)
            def _():
                for cp in out_cps(bi - 2, 0):
                    cp.wait()
            process(bi, 0)

        @pl.when(jnp.logical_not(even))
        def _():
            @pl.when(bi >= 2)
            def _():
                for cp in out_cps(bi - 2, 1):
                    cp.wait()
            process(bi, 1)

        return carry

    lax.fori_loop(0, BPW, body, 0)
    for cp in out_cps(BPW - 2, 0):
        cp.wait()
    for cp in out_cps(BPW - 1, 1):
        cp.wait()


def kernel(mcc_code, tr_type, merchant_id, amount, seq_lens, W_mcc, W_tr, W_mer):
    del seq_lens
    num = pl.pallas_call(
        _scaler_body,
        out_shape=jax.ShapeDtypeStruct((B, T), jnp.float32),
    )(amount)
    mcc1d = mcc_code.astype(jnp.int32).reshape(N)
    tr1d = tr_type.astype(jnp.int32).reshape(N)
    mer1d = merchant_id.astype(jnp.int32).reshape(N)
    stg = _sc_gather(mcc1d, tr1d, mer1d, num.reshape(N), W_mcc, W_tr, W_mer)
    planes = pl.pallas_call(
        _repack_body,
        grid=(TT, B // 512),
        in_specs=[pl.BlockSpec((512, RW), lambda ti, bj: (ti * (B // 512) + bj, 0))],
        out_specs=pl.BlockSpec((DO, 1, 4, 8, 128), lambda ti, bj: (0, ti, bj, 0, 0)),
        out_shape=jax.ShapeDtypeStruct((DO, TT, B // 128, 8, 128), jnp.float32),
    )(stg)
    return planes.transpose(2, 4, 1, 3, 0).reshape(B, T, DO)


# speedup vs baseline: 2.1668x; 1.0066x over previous
import functools

import jax
import jax.numpy as jnp
from jax import lax
from jax.experimental import pallas as pl
from jax.experimental.pallas import tpu as pltpu
from jax.experimental.pallas import tpu_sc as plsc

B, T = 1024, 200
N = B * T
V1, V2, V3 = 100000, 100000, 1000000
D1, D2, D3 = 32, 32, 16
DO = D1 + D2 + D3 + 1
EPS = 1e-5
SLOT = 96
TT = T // 8
RW = 8 * SLOT

NC, NS = 2, 16
NW = NC * NS
BPW = B // NW
SUBA, SUBB = 128, 72


def _scaler_body(a_ref, o_ref):
    x = a_ref[...]
    mean = jnp.mean(x)
    cx = x - mean
    var = jnp.mean(cx * cx)
    y = cx * lax.rsqrt(var + EPS)
    o_ref[...] = jnp.log1p(jnp.abs(y)) * jnp.sign(y)


def _repack_body(x_ref, o_ref):
    for h in range(8):
        for s in range(8):
            o_ref[:, 0, h, s, :] = (
                x_ref[pl.ds(h * 128, 128), pl.ds(s * SLOT, DO)].T)


_mesh = plsc.VectorSubcoreMesh(core_axis_name="c", subcore_axis_name="s")


@functools.partial(
    pl.kernel,
    mesh=_mesh,
    compiler_params=pltpu.CompilerParams(use_tc_tiling_on_sc=False),
    out_type=jax.ShapeDtypeStruct((TT * B, RW), jnp.float32),
    scratch_types=[
        pltpu.VMEM((2, 208), jnp.int32),
        pltpu.VMEM((2, 208), jnp.int32),
        pltpu.VMEM((2, 208), jnp.int32),
        pltpu.VMEM((2, SUBA, D1), jnp.float32),
        pltpu.VMEM((2, SUBB, D1), jnp.float32),
        pltpu.VMEM((2, SUBA, D2), jnp.float32),
        pltpu.VMEM((2, SUBB, D2), jnp.float32),
        pltpu.VMEM((2, SUBA, D3), jnp.float32),
        pltpu.VMEM((2, SUBB, D3), jnp.float32),
        pltpu.VMEM((2, 208), jnp.float32),
        pltpu.VMEM((2, TT, RW), jnp.float32),
        pltpu.SemaphoreType.DMA,
        pltpu.SemaphoreType.DMA,
        pltpu.SemaphoreType.DMA,
        pltpu.SemaphoreType.DMA,
    ],
)
def _sc_gather(mcc_hbm, tr_hbm, mer_hbm, num_hbm, wm_hbm, wt_hbm, we_hbm,
               out_hbm, idx1, idx2, idx3, r1a, r1b, r2a, r2b, r3a, r3b,
               numv, comb, sem0, sem1, osem0, osem1):
    wid = lax.axis_index("s") * NC + lax.axis_index("c")
    sems = (sem0, sem1)
    osems = (osem0, osem1)

    def gather_cps(bi, u):
        del bi
        sem = sems[u]
        return [
            pltpu.make_async_copy(
                wm_hbm.at[idx1.at[u, pl.ds(0, SUBA)]], r1a.at[u], sem),
            pltpu.make_async_copy(
                wt_hbm.at[idx2.at[u, pl.ds(0, SUBA)]], r2a.at[u], sem),
            pltpu.make_async_copy(
                we_hbm.at[idx3.at[u, pl.ds(0, SUBA)]], r3a.at[u], sem),
            pltpu.make_async_copy(
                wm_hbm.at[idx1.at[u, pl.ds(SUBA, SUBB)]], r1b.at[u], sem),
            pltpu.make_async_copy(
                wt_hbm.at[idx2.at[u, pl.ds(SUBA, SUBB)]], r2b.at[u], sem),
            pltpu.make_async_copy(
                we_hbm.at[idx3.at[u, pl.ds(SUBA, SUBB)]], r3b.at[u], sem),
        ]

    def stage_and_fire(bi, u):
        base = (wid * BPW + bi) * T
        stcps = [
            pltpu.make_async_copy(
                mcc_hbm.at[pl.ds(base, T)], idx1.at[u, pl.ds(0, T)], sems[u]),
            pltpu.make_async_copy(
                tr_hbm.at[pl.ds(base, T)], idx2.at[u, pl.ds(0, T)], sems[u]),
            pltpu.make_async_copy(
                mer_hbm.at[pl.ds(base, T)], idx3.at[u, pl.ds(0, T)], sems[u]),
            pltpu.make_async_copy(
                num_hbm.at[pl.ds(base, T)], numv.at[u, pl.ds(0, T)], sems[u]),
        ]
        for cp in stcps:
            cp.start()
        for cp in stcps:
            cp.wait()
        for cp in gather_cps(bi, u):
            cp.start()

    def out_cps(bi, u):
        b = wid * BPW + bi
        return [
            pltpu.make_async_copy(
                comb.at[u, pl.ds(ti, 1)],
                out_hbm.at[pl.ds(ti * B + b, 1)], osems[u])
            for ti in range(TT)
        ]

    def process(bi, u):
        for cp in gather_cps(bi, u):
            cp.wait()

        def emit(g, row0, x1, x2, x3):
            numvec = numv[u, pl.ds(g * 8, 16)]
            for j in range(8):
                i = row0 + j
                o = j * SLOT
                comb[u, g, pl.ds(o, 16)] = x1[u, i, pl.ds(0, 16)]
                comb[u, g, pl.ds(o + 16, 16)] = x1[u, i, pl.ds(16, 16)]
                comb[u, g, pl.ds(o + 32, 16)] = x2[u, i, pl.ds(0, 16)]
                comb[u, g, pl.ds(o + 48, 16)] = x2[u, i, pl.ds(16, 16)]
                comb[u, g, pl.ds(o + 65, 16)] = jnp.broadcast_to(
                    numvec[j], (16,))
                comb[u, g, pl.ds(o + 64, 16)] = x3[u, i, pl.ds(0, 16)]

        def grp_a(g, carry2):
            emit(g, g * 8, r1a, r2a, r3a)
            return carry2

        def grp_b(g, carry2):
            emit(g, g * 8 - SUBA, r1b, r2b, r3b)
            return carry2

        lax.fori_loop(0, SUBA // 8, grp_a, 0)
        lax.fori_loop(SUBA // 8, TT, grp_b, 0)
        for cp in out_cps(bi, u):
            cp.start()

    stage_and_fire(0, 0)

    def body(bi, carry):
        even = lax.rem(bi, 2) == 0
        more = bi + 1 < BPW

        @pl.when(jnp.logical_and(more, even))
        def _():
            stage_and_fire(bi + 1, 1)

        @pl.when(jnp.logical_and(more, jnp.logical_not(even)))
        def _():
            stage_and_fire(bi + 1, 0)

        @pl.when(even)
        def _():
            @pl.when(bi >= 2)
            def _():
                for cp in out_cps(bi - 2, 0):
                    cp.wait()
            process(bi, 0)

        @pl.when(jnp.logical_not(even))
        def _():
            @pl.when(bi >= 2)
            def _():
                for cp in out_cps(bi - 2, 1):
                    cp.wait()
            process(bi, 1)

        return carry

    lax.fori_loop(0, BPW, body, 0)
    for cp in out_cps(BPW - 2, 0):
        cp.wait()
    for cp in out_cps(BPW - 1, 1):
        cp.wait()


def kernel(mcc_code, tr_type, merchant_id, amount, seq_lens, W_mcc, W_tr, W_mer):
    del seq_lens
    num = pl.pallas_call(
        _scaler_body,
        out_shape=jax.ShapeDtypeStruct((B, T), jnp.float32),
    )(amount)
    mcc1d = mcc_code.astype(jnp.int32).reshape(N)
    tr1d = tr_type.astype(jnp.int32).reshape(N)
    mer1d = merchant_id.astype(jnp.int32).reshape(N)
    stg = _sc_gather(mcc1d, tr1d, mer1d, num.reshape(N), W_mcc, W_tr, W_mer)
    planes = pl.pallas_call(
        _repack_body,
        grid=(TT, B // 1024),
        in_specs=[pl.BlockSpec((1024, RW), lambda ti, bj: (ti * (B // 1024) + bj, 0))],
        out_specs=pl.BlockSpec((DO, 1, 8, 8, 128), lambda ti, bj: (0, ti, bj, 0, 0)),
        out_shape=jax.ShapeDtypeStruct((DO, TT, B // 128, 8, 128), jnp.float32),
    )(stg)
    return planes.transpose(2, 4, 1, 3, 0).reshape(B, T, DO)
